# baseline (speedup 1.0000x reference)
import functools
import jax
import jax.numpy as jnp
import numpy as np
from jax import lax
from jax.experimental import pallas as pl
from jax.experimental.pallas import tpu as pltpu
from jax.experimental.pallas import tpu_sc as plsc

B, TF, P, TA = 16, 4096, 512, 2048
MU_STAR = 350.0
KAPPA_STAR = 1.6
BETA = 1.0
EPS = 1e-8
OOD_MU = float(12.0 * np.log2(MU_STAR / 100.0))
OOD_KAPPA = float(12.0 * np.log2(KAPPA_STAR / 100.0))

NW = 32
ROWS_PER_W = (B * P) // NW
RBLK = 4
NBLK = ROWS_PER_W // RBLK
LANES = 16
RES_W = 3 * LANES


def _prep_body(f0_ref, d2_ref):
    f0 = f0_ref[...]
    col = lax.broadcasted_iota(jnp.int32, (B, TF), 1)
    colf = col.astype(jnp.float32)

    def sh_r(x, k, fill):
        return jnp.where(col < k, fill, pltpu.roll(x, k, axis=1))

    def sh_l(x, k, fill):
        return jnp.where(col >= TF - k, fill, pltpu.roll(x, TF - k, axis=1))

    voiced = f0 > 0.0
    vld0 = jnp.where(voiced, 1.0, 0.0)

    pval, pidx, pvld = f0, colf, vld0
    k = 1
    while k < TF:
        sv, si, sm = sh_r(pval, k, 0.0), sh_r(pidx, k, 0.0), sh_r(pvld, k, 0.0)
        keep = pvld > 0.0
        pval = jnp.where(keep, pval, sv)
        pidx = jnp.where(keep, pidx, si)
        pvld = jnp.maximum(pvld, sm)
        k *= 2
    nval, nidx, nvld = f0, colf, vld0
    k = 1
    while k < TF:
        sv, si, sm = sh_l(nval, k, 0.0), sh_l(nidx, k, 0.0), sh_l(nvld, k, 0.0)
        keep = nvld > 0.0
        nval = jnp.where(keep, nval, sv)
        nidx = jnp.where(keep, nidx, si)
        nvld = jnp.maximum(nvld, sm)
        k *= 2

    fv_val, fv_idx = nval[:, 0:1], nidx[:, 0:1]
    lv_val, lv_idx = pval[:, TF - 1:TF], pidx[:, TF - 1:TF]
    pv = pvld > 0.0
    nv = nvld > 0.0
    lo = jnp.where(pv, pidx, fv_idx)
    lo_v = jnp.where(pv, pval, fv_val)
    hi = jnp.where(nv, nidx, lv_idx)
    hi_v = jnp.where(nv, nval, lv_val)
    span = jnp.maximum(hi - lo, 1.0)
    wgt = jnp.clip((colf - lo) / span, 0.0, 1.0)
    fi = jnp.where(voiced, f0, lo_v + wgt * (hi_v - lo_v))

    def sh_r_edge(x, d):
        return jnp.where(col < d, x[:, 0:1], pltpu.roll(x, d, axis=1))

    def sh_l_edge(x, d):
        return jnp.where(col >= TF - d, x[:, TF - 1:TF], pltpu.roll(x, TF - d, axis=1))

    c0, c1, c2, c3 = -2.0 / 21.0, 3.0 / 21.0, 6.0 / 21.0, 7.0 / 21.0
    cs = (c3, c2, c1, c0)
    sm_ = c3 * fi
    for d in range(1, 4):
        sm_ = sm_ + cs[d] * (sh_r_edge(fi, d) + sh_l_edge(fi, d))
    d2 = jnp.abs(sh_r_edge(sm_, 1) - 2.0 * sm_ + sh_l_edge(sm_, 1))
    d2_ref[...] = d2


_prep = pl.pallas_call(
    _prep_body,
    out_shape=jax.ShapeDtypeStruct((B, TF), jnp.float32),
)


def _sc_body(attn_hbm, f0_hbm, d2_hbm, out_hbm, row_v, g1_v, g2_v, buf0_v, buf1_v,
             res_v, sem0, sem1):
    bb = lax.axis_index("s")
    half = lax.axis_index("c")
    wid = bb * 2 + half
    lane = lax.iota(jnp.int32, LANES)

    def build_pairsum(src_hbm, dst_ref):
        pltpu.sync_copy(src_hbm.at[bb], row_v)

        @pl.loop(0, TA // LANES)
        def _(i):
            idx_e = lane * 2 + i * (2 * LANES)
            ve = plsc.load_gather(row_v, [idx_e])
            vo = plsc.load_gather(row_v, [idx_e + 1])
            dst_ref[pl.ds(i * LANES, LANES)] = ve + vo

    build_pairsum(f0_hbm, g1_v)
    build_pairsum(d2_hbm, g2_v)

    row0 = bb * P + half * ROWS_PER_W

    def dma_block(i, buf, sem):
        return pltpu.make_async_copy(
            attn_hbm.at[pl.ds(row0 + i * RBLK, RBLK)], buf, sem
        )

    def compute_block(blk, buf):
        acc1 = [jnp.zeros((LANES,), jnp.float32) for _ in range(RBLK)]
        acc2 = [jnp.zeros((LANES,), jnp.float32) for _ in range(RBLK)]
        accs = [jnp.zeros((LANES,), jnp.float32) for _ in range(RBLK)]
        for j in range(TA // LANES):
            g1j = g1_v[pl.ds(j * LANES, LANES)]
            g2j = g2_v[pl.ds(j * LANES, LANES)]
            for r in range(RBLK):
                v = buf[r, pl.ds(j * LANES, LANES)]
                acc1[r] = acc1[r] + v * g1j
                acc2[r] = acc2[r] + v * g2j
                accs[r] = accs[r] + v
        for r in range(RBLK):
            base = (blk * RBLK + r) * RES_W
            res_v[pl.ds(base, LANES)] = acc1[r]
            res_v[pl.ds(base + LANES, LANES)] = acc2[r]
            res_v[pl.ds(base + 2 * LANES, LANES)] = accs[r]

    dma_block(0, buf0_v, sem0).start()

    @pl.loop(0, NBLK, step=2)
    def _(blk):
        dma_block(blk + 1, buf1_v, sem1).start()
        dma_block(blk, buf0_v, sem0).wait()
        compute_block(blk, buf0_v)

        @pl.when(blk + 2 < NBLK)
        def _():
            dma_block(blk + 2, buf0_v, sem0).start()

        dma_block(blk + 1, buf1_v, sem1).wait()
        compute_block(blk + 1, buf1_v)

    pltpu.sync_copy(res_v, out_hbm.at[wid])


@functools.cache
def _sc_reduce():
    return pl.kernel(
        _sc_body,
        out_type=jax.ShapeDtypeStruct((NW, ROWS_PER_W * RES_W), jnp.float32),
        mesh=plsc.VectorSubcoreMesh(
            core_axis_name="c", subcore_axis_name="s", num_cores=2, num_subcores=16
        ),
        compiler_params=pltpu.CompilerParams(needs_layout_passes=False),
        scratch_types=[
            pltpu.VMEM((TF,), jnp.float32),
            pltpu.VMEM((TA,), jnp.float32),
            pltpu.VMEM((TA,), jnp.float32),
            pltpu.VMEM((RBLK, TA), jnp.float32),
            pltpu.VMEM((RBLK, TA), jnp.float32),
            pltpu.VMEM((ROWS_PER_W * RES_W,), jnp.float32),
            pltpu.SemaphoreType.DMA,
            pltpu.SemaphoreType.DMA,
        ],
    )


def _fin_body(res_ref, uv_ref, w_ref, ess_ref):
    rec = res_ref[...]
    num1 = jnp.sum(rec[:, :, 0:LANES], axis=-1)
    num2 = jnp.sum(rec[:, :, LANES:2 * LANES], axis=-1)
    s = jnp.sum(rec[:, :, 2 * LANES:3 * LANES], axis=-1)
    uv = uv_ref[...]
    denom = jnp.maximum(2.0 * s, 1.0) + EPS
    pm = num1 / denom
    pc = num2 / denom
    inv_log2 = 1.0 / float(np.log(2.0))

    def raw_energy(pmetric, ood_s):
        pvv = pmetric * uv
        m = jnp.where(pvv != 0.0, 1.0, 0.0)
        semi = 12.0 * (jnp.log(jnp.maximum(pvv, 1e-6) / 100.0) * inv_log2)
        per = jnp.maximum(ood_s - semi, 0.0) * m
        cnt = jnp.sum(m, axis=1, keepdims=True)
        return jnp.sum(per, axis=1, keepdims=True) / jnp.maximum(cnt, 1.0)

    e = raw_energy(pm, OOD_MU) + raw_energy(pc, OOD_KAPPA)
    mu = jnp.mean(e)
    var = jnp.mean((e - mu) ** 2)
    sd = jnp.maximum(jnp.sqrt(var), 1e-6)
    z = (e - mu) / sd
    x = -BETA * z
    xm = jnp.max(x)
    ex = jnp.exp(x - xm)
    w = ex / jnp.sum(ex)
    w_ref[...] = float(B) * w
    ess_ref[...] = 1.0 / jnp.sum(w * w, axis=0, keepdims=True)


_finish = pl.pallas_call(
    _fin_body,
    out_shape=(
        jax.ShapeDtypeStruct((B, 1), jnp.float32),
        jax.ShapeDtypeStruct((1, 1), jnp.float32),
    ),
)


def kernel(f0_gd_frame, f2p_attn_gd, uv_mask):
    d2 = _prep(f0_gd_frame)
    attn2d = f2p_attn_gd.reshape(B * P, TA)
    res = _sc_reduce()(attn2d, f0_gd_frame, d2)
    rec = res.reshape(B, P, RES_W)
    w_col, ess11 = _finish(rec, uv_mask)
    return w_col.reshape(B), ess11.reshape(())


# speedup vs baseline: 1.0448x; 1.0448x over previous
import functools
import jax
import jax.numpy as jnp
import numpy as np
from jax import lax
from jax.experimental import pallas as pl
from jax.experimental.pallas import tpu as pltpu
from jax.experimental.pallas import tpu_sc as plsc

B, TF, P, TA = 16, 4096, 512, 2048
MU_STAR = 350.0
KAPPA_STAR = 1.6
BETA = 1.0
EPS = 1e-8
OOD_MU = float(12.0 * np.log2(MU_STAR / 100.0))
OOD_KAPPA = float(12.0 * np.log2(KAPPA_STAR / 100.0))

NW = 32
ROWS_PER_W = (B * P) // NW
RBLK = 8
NBLK = ROWS_PER_W // RBLK
LANES = 16
RES_W = 3 * LANES


def _prep_body(f0_ref, d2_ref):
    f0 = f0_ref[...]
    col = lax.broadcasted_iota(jnp.int32, (B, TF), 1)
    colf = col.astype(jnp.float32)

    def sh_r(x, k, fill):
        return jnp.where(col < k, fill, pltpu.roll(x, k, axis=1))

    def sh_l(x, k, fill):
        return jnp.where(col >= TF - k, fill, pltpu.roll(x, TF - k, axis=1))

    voiced = f0 > 0.0
    vld0 = jnp.where(voiced, 1.0, 0.0)

    pval, pidx, pvld = f0, colf, vld0
    k = 1
    while k < TF:
        sv, si, sm = sh_r(pval, k, 0.0), sh_r(pidx, k, 0.0), sh_r(pvld, k, 0.0)
        keep = pvld > 0.0
        pval = jnp.where(keep, pval, sv)
        pidx = jnp.where(keep, pidx, si)
        pvld = jnp.maximum(pvld, sm)
        k *= 2
    nval, nidx, nvld = f0, colf, vld0
    k = 1
    while k < TF:
        sv, si, sm = sh_l(nval, k, 0.0), sh_l(nidx, k, 0.0), sh_l(nvld, k, 0.0)
        keep = nvld > 0.0
        nval = jnp.where(keep, nval, sv)
        nidx = jnp.where(keep, nidx, si)
        nvld = jnp.maximum(nvld, sm)
        k *= 2

    fv_val, fv_idx = nval[:, 0:1], nidx[:, 0:1]
    lv_val, lv_idx = pval[:, TF - 1:TF], pidx[:, TF - 1:TF]
    pv = pvld > 0.0
    nv = nvld > 0.0
    lo = jnp.where(pv, pidx, fv_idx)
    lo_v = jnp.where(pv, pval, fv_val)
    hi = jnp.where(nv, nidx, lv_idx)
    hi_v = jnp.where(nv, nval, lv_val)
    span = jnp.maximum(hi - lo, 1.0)
    wgt = jnp.clip((colf - lo) / span, 0.0, 1.0)
    fi = jnp.where(voiced, f0, lo_v + wgt * (hi_v - lo_v))

    def sh_r_edge(x, d):
        return jnp.where(col < d, x[:, 0:1], pltpu.roll(x, d, axis=1))

    def sh_l_edge(x, d):
        return jnp.where(col >= TF - d, x[:, TF - 1:TF], pltpu.roll(x, TF - d, axis=1))

    c0, c1, c2, c3 = -2.0 / 21.0, 3.0 / 21.0, 6.0 / 21.0, 7.0 / 21.0
    cs = (c3, c2, c1, c0)
    sm_ = c3 * fi
    for d in range(1, 4):
        sm_ = sm_ + cs[d] * (sh_r_edge(fi, d) + sh_l_edge(fi, d))
    d2 = jnp.abs(sh_r_edge(sm_, 1) - 2.0 * sm_ + sh_l_edge(sm_, 1))
    d2_ref[...] = d2


_prep = pl.pallas_call(
    _prep_body,
    out_shape=jax.ShapeDtypeStruct((B, TF), jnp.float32),
)


def _sc_body(attn_hbm, f0_hbm, d2_hbm, out_hbm, row_v, g1_v, g2_v, abufs_v,
             res_v, dsems):
    bb = lax.axis_index("s")
    half = lax.axis_index("c")
    wid = bb * 2 + half
    lane = lax.iota(jnp.int32, LANES)

    def build_pairsum(src_hbm, dst_ref):
        pltpu.sync_copy(src_hbm.at[bb], row_v)

        @pl.loop(0, TA // LANES)
        def _(i):
            idx_e = lane * 2 + i * (2 * LANES)
            ve = plsc.load_gather(row_v, [idx_e])
            vo = plsc.load_gather(row_v, [idx_e + 1])
            dst_ref[pl.ds(i * LANES, LANES)] = ve + vo

    build_pairsum(f0_hbm, g1_v)
    build_pairsum(d2_hbm, g2_v)

    row0 = bb * P + half * ROWS_PER_W

    def dma_block(i, par):
        return pltpu.make_async_copy(
            attn_hbm.at[pl.ds(row0 + i * RBLK, RBLK)], abufs_v.at[par], dsems.at[par]
        )

    dma_block(0, 0).start()

    @pl.loop(0, NBLK)
    def _(blk):
        par = lax.rem(blk, 2)
        nxt = 1 - par

        @pl.when(blk + 1 < NBLK)
        def _():
            dma_block(blk + 1, nxt).start()

        dma_block(blk, par).wait()
        acc1 = [jnp.zeros((LANES,), jnp.float32) for _ in range(RBLK)]
        acc2 = [jnp.zeros((LANES,), jnp.float32) for _ in range(RBLK)]
        accs = [jnp.zeros((LANES,), jnp.float32) for _ in range(RBLK)]
        for j in range(TA // LANES):
            g1j = g1_v[pl.ds(j * LANES, LANES)]
            g2j = g2_v[pl.ds(j * LANES, LANES)]
            for r in range(RBLK):
                v = abufs_v[par, r, pl.ds(j * LANES, LANES)]
                acc1[r] = acc1[r] + v * g1j
                acc2[r] = acc2[r] + v * g2j
                accs[r] = accs[r] + v
        for r in range(RBLK):
            base = (blk * RBLK + r) * RES_W
            res_v[pl.ds(base, LANES)] = acc1[r]
            res_v[pl.ds(base + LANES, LANES)] = acc2[r]
            res_v[pl.ds(base + 2 * LANES, LANES)] = accs[r]

    pltpu.sync_copy(res_v, out_hbm.at[wid])


@functools.cache
def _sc_reduce():
    return pl.kernel(
        _sc_body,
        out_type=jax.ShapeDtypeStruct((NW, ROWS_PER_W * RES_W), jnp.float32),
        mesh=plsc.VectorSubcoreMesh(
            core_axis_name="c", subcore_axis_name="s", num_cores=2, num_subcores=16
        ),
        compiler_params=pltpu.CompilerParams(needs_layout_passes=False),
        scratch_types=[
            pltpu.VMEM((TF,), jnp.float32),
            pltpu.VMEM((TA,), jnp.float32),
            pltpu.VMEM((TA,), jnp.float32),
            pltpu.VMEM((2, RBLK, TA), jnp.float32),
            pltpu.VMEM((ROWS_PER_W * RES_W,), jnp.float32),
            pltpu.SemaphoreType.DMA((2,)),
        ],
    )


def _fin_body(res_ref, uv_ref, w_ref, ess_ref):
    rec = res_ref[...]
    num1 = jnp.sum(rec[:, :, 0:LANES], axis=-1)
    num2 = jnp.sum(rec[:, :, LANES:2 * LANES], axis=-1)
    s = jnp.sum(rec[:, :, 2 * LANES:3 * LANES], axis=-1)
    uv = uv_ref[...]
    denom = jnp.maximum(2.0 * s, 1.0) + EPS
    pm = num1 / denom
    pc = num2 / denom
    inv_log2 = 1.0 / float(np.log(2.0))

    def raw_energy(pmetric, ood_s):
        pvv = pmetric * uv
        m = jnp.where(pvv != 0.0, 1.0, 0.0)
        semi = 12.0 * (jnp.log(jnp.maximum(pvv, 1e-6) / 100.0) * inv_log2)
        per = jnp.maximum(ood_s - semi, 0.0) * m
        cnt = jnp.sum(m, axis=1, keepdims=True)
        return jnp.sum(per, axis=1, keepdims=True) / jnp.maximum(cnt, 1.0)

    e = raw_energy(pm, OOD_MU) + raw_energy(pc, OOD_KAPPA)
    mu = jnp.mean(e)
    var = jnp.mean((e - mu) ** 2)
    sd = jnp.maximum(jnp.sqrt(var), 1e-6)
    z = (e - mu) / sd
    x = -BETA * z
    xm = jnp.max(x)
    ex = jnp.exp(x - xm)
    w = ex / jnp.sum(ex)
    w_ref[...] = float(B) * w
    ess_ref[...] = 1.0 / jnp.sum(w * w, axis=0, keepdims=True)


_finish = pl.pallas_call(
    _fin_body,
    out_shape=(
        jax.ShapeDtypeStruct((B, 1), jnp.float32),
        jax.ShapeDtypeStruct((1, 1), jnp.float32),
    ),
)


def kernel(f0_gd_frame, f2p_attn_gd, uv_mask):
    d2 = _prep(f0_gd_frame)
    attn2d = f2p_attn_gd.reshape(B * P, TA)
    res = _sc_reduce()(attn2d, f0_gd_frame, d2)
    rec = res.reshape(B, P, RES_W)
    w_col, ess11 = _finish(rec, uv_mask)
    return w_col.reshape(B), ess11.reshape(())


# speedup vs baseline: 1.1740x; 1.1236x over previous
import functools
import jax
import jax.numpy as jnp
import numpy as np
from jax import lax
from jax.experimental import pallas as pl
from jax.experimental.pallas import tpu as pltpu
from jax.experimental.pallas import tpu_sc as plsc

B, TF, P, TA = 16, 4096, 512, 2048
MU_STAR = 350.0
KAPPA_STAR = 1.6
BETA = 1.0
EPS = 1e-8
OOD_MU = float(12.0 * np.log2(MU_STAR / 100.0))
OOD_KAPPA = float(12.0 * np.log2(KAPPA_STAR / 100.0))

NW = 32
TB = 12
NB_SC = B - TB
WPB = NW // NB_SC
ROWS_PER_W = P // WPB
RBLK = 8
NBLK = ROWS_PER_W // RBLK
LANES = 16
RES_W = 3 * LANES


def _prep_body(f0_ref, d2_ref):
    f0 = f0_ref[...]
    col = lax.broadcasted_iota(jnp.int32, (B, TF), 1)
    colf = col.astype(jnp.float32)

    def sh_r(x, k, fill):
        return jnp.where(col < k, fill, pltpu.roll(x, k, axis=1))

    def sh_l(x, k, fill):
        return jnp.where(col >= TF - k, fill, pltpu.roll(x, TF - k, axis=1))

    voiced = f0 > 0.0
    vld0 = jnp.where(voiced, 1.0, 0.0)

    pval, pidx, pvld = f0, colf, vld0
    k = 1
    while k < TF:
        sv, si, sm = sh_r(pval, k, 0.0), sh_r(pidx, k, 0.0), sh_r(pvld, k, 0.0)
        keep = pvld > 0.0
        pval = jnp.where(keep, pval, sv)
        pidx = jnp.where(keep, pidx, si)
        pvld = jnp.maximum(pvld, sm)
        k *= 2
    nval, nidx, nvld = f0, colf, vld0
    k = 1
    while k < TF:
        sv, si, sm = sh_l(nval, k, 0.0), sh_l(nidx, k, 0.0), sh_l(nvld, k, 0.0)
        keep = nvld > 0.0
        nval = jnp.where(keep, nval, sv)
        nidx = jnp.where(keep, nidx, si)
        nvld = jnp.maximum(nvld, sm)
        k *= 2

    fv_val, fv_idx = nval[:, 0:1], nidx[:, 0:1]
    lv_val, lv_idx = pval[:, TF - 1:TF], pidx[:, TF - 1:TF]
    pv = pvld > 0.0
    nv = nvld > 0.0
    lo = jnp.where(pv, pidx, fv_idx)
    lo_v = jnp.where(pv, pval, fv_val)
    hi = jnp.where(nv, nidx, lv_idx)
    hi_v = jnp.where(nv, nval, lv_val)
    span = jnp.maximum(hi - lo, 1.0)
    wgt = jnp.clip((colf - lo) / span, 0.0, 1.0)
    fi = jnp.where(voiced, f0, lo_v + wgt * (hi_v - lo_v))

    def sh_r_edge(x, d):
        return jnp.where(col < d, x[:, 0:1], pltpu.roll(x, d, axis=1))

    def sh_l_edge(x, d):
        return jnp.where(col >= TF - d, x[:, TF - 1:TF], pltpu.roll(x, TF - d, axis=1))

    c0, c1, c2, c3 = -2.0 / 21.0, 3.0 / 21.0, 6.0 / 21.0, 7.0 / 21.0
    cs = (c3, c2, c1, c0)
    sm_ = c3 * fi
    for d in range(1, 4):
        sm_ = sm_ + cs[d] * (sh_r_edge(fi, d) + sh_l_edge(fi, d))
    d2 = jnp.abs(sh_r_edge(sm_, 1) - 2.0 * sm_ + sh_l_edge(sm_, 1))
    d2_ref[...] = d2


_prep = pl.pallas_call(
    _prep_body,
    out_shape=jax.ShapeDtypeStruct((B, TF), jnp.float32),
)


def _tc_body(attn_ref, f0v_ref, d2v_ref, out_ref):
    g1 = jnp.sum(f0v_ref[0], axis=1)
    g2 = jnp.sum(d2v_ref[0], axis=1)
    colg = lax.broadcasted_iota(jnp.int32, (TA, 8), 1)
    g = jnp.where(colg == 0, g1[:, None],
                  jnp.where(colg == 1, g2[:, None],
                            jnp.where(colg == 2, 1.0, 0.0)))
    y = jnp.dot(attn_ref[0], g, precision=jax.lax.Precision.HIGHEST,
                preferred_element_type=jnp.float32)
    out_ref[0] = y


_tc_num = pl.pallas_call(
    _tc_body,
    grid=(TB,),
    in_specs=[
        pl.BlockSpec((1, P, TA), lambda b: (b, 0, 0)),
        pl.BlockSpec((1, TA, 2), lambda b: (b, 0, 0)),
        pl.BlockSpec((1, TA, 2), lambda b: (b, 0, 0)),
    ],
    out_specs=pl.BlockSpec((1, P, 8), lambda b: (b, 0, 0)),
    out_shape=jax.ShapeDtypeStruct((TB, P, 8), jnp.float32),
)


def _sc_body(attn_hbm, f0_hbm, d2_hbm, out_hbm, row_v, g1_v, g2_v, abufs_v,
             res_v, dsems):
    wid = lax.axis_index("s") * 2 + lax.axis_index("c")
    bb = TB + wid // WPB
    seg = lax.rem(wid, WPB)
    lane = lax.iota(jnp.int32, LANES)

    def build_pairsum(src_hbm, dst_ref):
        pltpu.sync_copy(src_hbm.at[bb], row_v)

        @pl.loop(0, TA // LANES)
        def _(i):
            idx_e = lane * 2 + i * (2 * LANES)
            ve = plsc.load_gather(row_v, [idx_e])
            vo = plsc.load_gather(row_v, [idx_e + 1])
            dst_ref[pl.ds(i * LANES, LANES)] = ve + vo

    build_pairsum(f0_hbm, g1_v)
    build_pairsum(d2_hbm, g2_v)

    row0 = bb * P + seg * ROWS_PER_W

    def dma_block(i, par):
        return pltpu.make_async_copy(
            attn_hbm.at[pl.ds(row0 + i * RBLK, RBLK)], abufs_v.at[par], dsems.at[par]
        )

    dma_block(0, 0).start()

    @pl.loop(0, NBLK)
    def _(blk):
        par = lax.rem(blk, 2)
        nxt = 1 - par

        @pl.when(blk + 1 < NBLK)
        def _():
            dma_block(blk + 1, nxt).start()

        dma_block(blk, par).wait()
        acc1 = [jnp.zeros((LANES,), jnp.float32) for _ in range(RBLK)]
        acc2 = [jnp.zeros((LANES,), jnp.float32) for _ in range(RBLK)]
        accs = [jnp.zeros((LANES,), jnp.float32) for _ in range(RBLK)]
        for j in range(TA // LANES):
            g1j = g1_v[pl.ds(j * LANES, LANES)]
            g2j = g2_v[pl.ds(j * LANES, LANES)]
            for r in range(RBLK):
                v = abufs_v[par, r, pl.ds(j * LANES, LANES)]
                acc1[r] = acc1[r] + v * g1j
                acc2[r] = acc2[r] + v * g2j
                accs[r] = accs[r] + v
        for r in range(RBLK):
            base = (blk * RBLK + r) * RES_W
            res_v[pl.ds(base, LANES)] = acc1[r]
            res_v[pl.ds(base + LANES, LANES)] = acc2[r]
            res_v[pl.ds(base + 2 * LANES, LANES)] = accs[r]

    pltpu.sync_copy(res_v, out_hbm.at[wid])


@functools.cache
def _sc_reduce():
    return pl.kernel(
        _sc_body,
        out_type=jax.ShapeDtypeStruct((NW, ROWS_PER_W * RES_W), jnp.float32),
        mesh=plsc.VectorSubcoreMesh(
            core_axis_name="c", subcore_axis_name="s", num_cores=2, num_subcores=16
        ),
        compiler_params=pltpu.CompilerParams(needs_layout_passes=False),
        scratch_types=[
            pltpu.VMEM((TF,), jnp.float32),
            pltpu.VMEM((TA,), jnp.float32),
            pltpu.VMEM((TA,), jnp.float32),
            pltpu.VMEM((2, RBLK, TA), jnp.float32),
            pltpu.VMEM((ROWS_PER_W * RES_W,), jnp.float32),
            pltpu.SemaphoreType.DMA((2,)),
        ],
    )


def _fin_body(tc_ref, res_ref, uv_ref, w_ref, ess_ref):
    tc = tc_ref[...]
    rec = res_ref[...]
    num1 = jnp.concatenate(
        [tc[:, :, 0], jnp.sum(rec[:, :, 0:LANES], axis=-1)], axis=0)
    num2 = jnp.concatenate(
        [tc[:, :, 1], jnp.sum(rec[:, :, LANES:2 * LANES], axis=-1)], axis=0)
    s = jnp.concatenate(
        [tc[:, :, 2], jnp.sum(rec[:, :, 2 * LANES:3 * LANES], axis=-1)], axis=0)
    uv = uv_ref[...]
    denom = jnp.maximum(2.0 * s, 1.0) + EPS
    pm = num1 / denom
    pc = num2 / denom
    inv_log2 = 1.0 / float(np.log(2.0))

    def raw_energy(pmetric, ood_s):
        pvv = pmetric * uv
        m = jnp.where(pvv != 0.0, 1.0, 0.0)
        semi = 12.0 * (jnp.log(jnp.maximum(pvv, 1e-6) / 100.0) * inv_log2)
        per = jnp.maximum(ood_s - semi, 0.0) * m
        cnt = jnp.sum(m, axis=1, keepdims=True)
        return jnp.sum(per, axis=1, keepdims=True) / jnp.maximum(cnt, 1.0)

    e = raw_energy(pm, OOD_MU) + raw_energy(pc, OOD_KAPPA)
    mu = jnp.mean(e)
    var = jnp.mean((e - mu) ** 2)
    sd = jnp.maximum(jnp.sqrt(var), 1e-6)
    z = (e - mu) / sd
    x = -BETA * z
    xm = jnp.max(x)
    ex = jnp.exp(x - xm)
    w = ex / jnp.sum(ex)
    w_ref[...] = float(B) * w
    ess_ref[...] = 1.0 / jnp.sum(w * w, axis=0, keepdims=True)


_finish = pl.pallas_call(
    _fin_body,
    out_shape=(
        jax.ShapeDtypeStruct((B, 1), jnp.float32),
        jax.ShapeDtypeStruct((1, 1), jnp.float32),
    ),
)


def kernel(f0_gd_frame, f2p_attn_gd, uv_mask):
    d2 = _prep(f0_gd_frame)
    attn2d = f2p_attn_gd.reshape(B * P, TA)
    res = _sc_reduce()(attn2d, f0_gd_frame, d2)
    f0v = f0_gd_frame.reshape(B, TA, 2)
    d2v = d2.reshape(B, TA, 2)
    tcnum = _tc_num(f2p_attn_gd, f0v, d2v)
    rec = res.reshape(NB_SC, P, RES_W)
    w_col, ess11 = _finish(tcnum, rec, uv_mask)
    return w_col.reshape(B), ess11.reshape(())


# speedup vs baseline: 1.4143x; 1.2047x over previous
import functools
import jax
import jax.numpy as jnp
import numpy as np
from jax import lax
from jax.experimental import pallas as pl
from jax.experimental.pallas import tpu as pltpu
from jax.experimental.pallas import tpu_sc as plsc

B, TF, P, TA = 16, 4096, 512, 2048
MU_STAR = 350.0
KAPPA_STAR = 1.6
BETA = 1.0
EPS = 1e-8
OOD_MU = float(12.0 * np.log2(MU_STAR / 100.0))
OOD_KAPPA = float(12.0 * np.log2(KAPPA_STAR / 100.0))

NW = 32
TB = 12
NB_SC = B - TB
WPB = NW // NB_SC
ROWS_PER_W = P // WPB
RBLK = 8
NBLK = ROWS_PER_W // RBLK
LANES = 16
RES_W = 3 * LANES


def _prep_body(f0_ref, d2_ref):
    f0 = f0_ref[...]
    col = lax.broadcasted_iota(jnp.int32, (B, TF), 1)
    colf = col.astype(jnp.float32)

    def sh_r(x, k, fill):
        return jnp.where(col < k, fill, pltpu.roll(x, k, axis=1))

    def sh_l(x, k, fill):
        return jnp.where(col >= TF - k, fill, pltpu.roll(x, TF - k, axis=1))

    voiced = f0 > 0.0
    vld0 = jnp.where(voiced, 1.0, 0.0)

    pval, pidx, pvld = f0, colf, vld0
    k = 1
    while k < TF:
        sv, si, sm = sh_r(pval, k, 0.0), sh_r(pidx, k, 0.0), sh_r(pvld, k, 0.0)
        keep = pvld > 0.0
        pval = jnp.where(keep, pval, sv)
        pidx = jnp.where(keep, pidx, si)
        pvld = jnp.maximum(pvld, sm)
        k *= 2
    nval, nidx, nvld = f0, colf, vld0
    k = 1
    while k < TF:
        sv, si, sm = sh_l(nval, k, 0.0), sh_l(nidx, k, 0.0), sh_l(nvld, k, 0.0)
        keep = nvld > 0.0
        nval = jnp.where(keep, nval, sv)
        nidx = jnp.where(keep, nidx, si)
        nvld = jnp.maximum(nvld, sm)
        k *= 2

    fv_val, fv_idx = nval[:, 0:1], nidx[:, 0:1]
    lv_val, lv_idx = pval[:, TF - 1:TF], pidx[:, TF - 1:TF]
    pv = pvld > 0.0
    nv = nvld > 0.0
    lo = jnp.where(pv, pidx, fv_idx)
    lo_v = jnp.where(pv, pval, fv_val)
    hi = jnp.where(nv, nidx, lv_idx)
    hi_v = jnp.where(nv, nval, lv_val)
    span = jnp.maximum(hi - lo, 1.0)
    wgt = jnp.clip((colf - lo) / span, 0.0, 1.0)
    fi = jnp.where(voiced, f0, lo_v + wgt * (hi_v - lo_v))

    def sh_r_edge(x, d):
        return jnp.where(col < d, x[:, 0:1], pltpu.roll(x, d, axis=1))

    def sh_l_edge(x, d):
        return jnp.where(col >= TF - d, x[:, TF - 1:TF], pltpu.roll(x, TF - d, axis=1))

    c0, c1, c2, c3 = -2.0 / 21.0, 3.0 / 21.0, 6.0 / 21.0, 7.0 / 21.0
    cs = (c3, c2, c1, c0)
    sm_ = c3 * fi
    for d in range(1, 4):
        sm_ = sm_ + cs[d] * (sh_r_edge(fi, d) + sh_l_edge(fi, d))
    d2 = jnp.abs(sh_r_edge(sm_, 1) - 2.0 * sm_ + sh_l_edge(sm_, 1))
    d2_ref[...] = d2


_prep = pl.pallas_call(
    _prep_body,
    out_shape=jax.ShapeDtypeStruct((B, TF), jnp.float32),
)


def _tc_body(attn_ref, f0v_ref, d2v_ref, out_ref):
    g1 = jnp.sum(f0v_ref[0], axis=1)
    g2 = jnp.sum(d2v_ref[0], axis=1)
    a = attn_ref[0]
    num1 = jnp.sum(a * g1[None, :], axis=1, keepdims=True)
    num2 = jnp.sum(a * g2[None, :], axis=1, keepdims=True)
    s = jnp.sum(a, axis=1, keepdims=True)
    colg = lax.broadcasted_iota(jnp.int32, (P, 8), 1)
    out_ref[0] = jnp.where(colg == 0, num1,
                           jnp.where(colg == 1, num2,
                                     jnp.where(colg == 2, s, 0.0)))


_tc_num = pl.pallas_call(
    _tc_body,
    grid=(TB,),
    in_specs=[
        pl.BlockSpec((1, P, TA), lambda b: (b, 0, 0)),
        pl.BlockSpec((1, TA, 2), lambda b: (b, 0, 0)),
        pl.BlockSpec((1, TA, 2), lambda b: (b, 0, 0)),
    ],
    out_specs=pl.BlockSpec((1, P, 8), lambda b: (b, 0, 0)),
    out_shape=jax.ShapeDtypeStruct((TB, P, 8), jnp.float32),
)


def _sc_body(attn_hbm, f0_hbm, d2_hbm, out_hbm, row_v, g1_v, g2_v, abufs_v,
             res_v, dsems):
    wid = lax.axis_index("s") * 2 + lax.axis_index("c")
    bb = TB + wid // WPB
    seg = lax.rem(wid, WPB)
    lane = lax.iota(jnp.int32, LANES)

    def build_pairsum(src_hbm, dst_ref):
        pltpu.sync_copy(src_hbm.at[bb], row_v)

        @pl.loop(0, TA // LANES)
        def _(i):
            idx_e = lane * 2 + i * (2 * LANES)
            ve = plsc.load_gather(row_v, [idx_e])
            vo = plsc.load_gather(row_v, [idx_e + 1])
            dst_ref[pl.ds(i * LANES, LANES)] = ve + vo

    build_pairsum(f0_hbm, g1_v)
    build_pairsum(d2_hbm, g2_v)

    row0 = bb * P + seg * ROWS_PER_W

    def dma_block(i, par):
        return pltpu.make_async_copy(
            attn_hbm.at[pl.ds(row0 + i * RBLK, RBLK)], abufs_v.at[par], dsems.at[par]
        )

    dma_block(0, 0).start()

    @pl.loop(0, NBLK)
    def _(blk):
        par = lax.rem(blk, 2)
        nxt = 1 - par

        @pl.when(blk + 1 < NBLK)
        def _():
            dma_block(blk + 1, nxt).start()

        dma_block(blk, par).wait()
        acc1 = [jnp.zeros((LANES,), jnp.float32) for _ in range(RBLK)]
        acc2 = [jnp.zeros((LANES,), jnp.float32) for _ in range(RBLK)]
        accs = [jnp.zeros((LANES,), jnp.float32) for _ in range(RBLK)]
        for j in range(TA // LANES):
            g1j = g1_v[pl.ds(j * LANES, LANES)]
            g2j = g2_v[pl.ds(j * LANES, LANES)]
            for r in range(RBLK):
                v = abufs_v[par, r, pl.ds(j * LANES, LANES)]
                acc1[r] = acc1[r] + v * g1j
                acc2[r] = acc2[r] + v * g2j
                accs[r] = accs[r] + v
        for r in range(RBLK):
            base = (blk * RBLK + r) * RES_W
            res_v[pl.ds(base, LANES)] = acc1[r]
            res_v[pl.ds(base + LANES, LANES)] = acc2[r]
            res_v[pl.ds(base + 2 * LANES, LANES)] = accs[r]

    pltpu.sync_copy(res_v, out_hbm.at[wid])


@functools.cache
def _sc_reduce():
    return pl.kernel(
        _sc_body,
        out_type=jax.ShapeDtypeStruct((NW, ROWS_PER_W * RES_W), jnp.float32),
        mesh=plsc.VectorSubcoreMesh(
            core_axis_name="c", subcore_axis_name="s", num_cores=2, num_subcores=16
        ),
        compiler_params=pltpu.CompilerParams(needs_layout_passes=False),
        scratch_types=[
            pltpu.VMEM((TF,), jnp.float32),
            pltpu.VMEM((TA,), jnp.float32),
            pltpu.VMEM((TA,), jnp.float32),
            pltpu.VMEM((2, RBLK, TA), jnp.float32),
            pltpu.VMEM((ROWS_PER_W * RES_W,), jnp.float32),
            pltpu.SemaphoreType.DMA((2,)),
        ],
    )


def _fin_body(tc_ref, res_ref, uv_ref, w_ref, ess_ref):
    tc = tc_ref[...]
    rec = res_ref[...]
    num1 = jnp.concatenate(
        [tc[:, :, 0], jnp.sum(rec[:, :, 0:LANES], axis=-1)], axis=0)
    num2 = jnp.concatenate(
        [tc[:, :, 1], jnp.sum(rec[:, :, LANES:2 * LANES], axis=-1)], axis=0)
    s = jnp.concatenate(
        [tc[:, :, 2], jnp.sum(rec[:, :, 2 * LANES:3 * LANES], axis=-1)], axis=0)
    uv = uv_ref[...]
    denom = jnp.maximum(2.0 * s, 1.0) + EPS
    pm = num1 / denom
    pc = num2 / denom
    inv_log2 = 1.0 / float(np.log(2.0))

    def raw_energy(pmetric, ood_s):
        pvv = pmetric * uv
        m = jnp.where(pvv != 0.0, 1.0, 0.0)
        semi = 12.0 * (jnp.log(jnp.maximum(pvv, 1e-6) / 100.0) * inv_log2)
        per = jnp.maximum(ood_s - semi, 0.0) * m
        cnt = jnp.sum(m, axis=1, keepdims=True)
        return jnp.sum(per, axis=1, keepdims=True) / jnp.maximum(cnt, 1.0)

    e = raw_energy(pm, OOD_MU) + raw_energy(pc, OOD_KAPPA)
    mu = jnp.mean(e)
    var = jnp.mean((e - mu) ** 2)
    sd = jnp.maximum(jnp.sqrt(var), 1e-6)
    z = (e - mu) / sd
    x = -BETA * z
    xm = jnp.max(x)
    ex = jnp.exp(x - xm)
    w = ex / jnp.sum(ex)
    w_ref[...] = float(B) * w
    ess_ref[...] = 1.0 / jnp.sum(w * w, axis=0, keepdims=True)


_finish = pl.pallas_call(
    _fin_body,
    out_shape=(
        jax.ShapeDtypeStruct((B, 1), jnp.float32),
        jax.ShapeDtypeStruct((1, 1), jnp.float32),
    ),
)


def kernel(f0_gd_frame, f2p_attn_gd, uv_mask):
    d2 = _prep(f0_gd_frame)
    attn2d = f2p_attn_gd.reshape(B * P, TA)
    res = _sc_reduce()(attn2d, f0_gd_frame, d2)
    f0v = f0_gd_frame.reshape(B, TA, 2)
    d2v = d2.reshape(B, TA, 2)
    tcnum = _tc_num(f2p_attn_gd, f0v, d2v)
    rec = res.reshape(NB_SC, P, RES_W)
    w_col, ess11 = _finish(tcnum, rec, uv_mask)
    return w_col.reshape(B), ess11.reshape(())
